# Initial kernel scaffold; baseline (speedup 1.0000x reference)
import jax
import jax.numpy as jnp
from jax.experimental import pallas as pl


def kernel(lookup, table):
    raise NotImplementedError("write your pallas kernel here")


# speedup vs baseline: 4.1352x; 4.1352x over previous
import functools

import jax
import jax.numpy as jnp
from jax import lax
from jax.experimental import pallas as pl
from jax.experimental.pallas import tpu as pltpu
from jax.experimental.pallas import tpu_sc as plsc

VEC = 128
K = 8


@functools.partial(jax.jit, static_argnames=())
def kernel(lookup, table):
    B, T = lookup.shape
    V, D = table.shape
    N = B * T

    info = plsc.get_sparse_core_info()
    NW = info.num_cores * info.num_subcores
    assert N % (NW * VEC * K) == 0

    vecs_per_w = N // (NW * VEC)
    iters = vecs_per_w // K
    slab_rows = K * VEC

    idx2d = lookup.reshape(N // VEC, VEC).astype(jnp.int32)

    mesh = plsc.VectorSubcoreMesh(core_axis_name="c", subcore_axis_name="s")

    @functools.partial(
        pl.kernel,
        mesh=mesh,
        out_type=jax.ShapeDtypeStruct((N, D), jnp.float32),
        scratch_types=[
            pltpu.VMEM((K, VEC), jnp.int32),
            pltpu.VMEM((slab_rows, D), jnp.float32),
            pltpu.SemaphoreType.DMA,
        ],
        compiler_params=pltpu.CompilerParams(use_tc_tiling_on_sc=False),
    )
    def gather_kernel(table_hbm, idx_hbm, out_hbm, idx_v, rows_v, sem):
        wid = lax.axis_index("s") * info.num_cores + lax.axis_index("c")
        vrow0 = wid * vecs_per_w
        row0 = wid * (vecs_per_w * VEC)

        def body(i, _):
            pltpu.sync_copy(idx_hbm.at[pl.ds(vrow0 + i * K, K)], idx_v)
            copies = [
                pltpu.async_copy(
                    table_hbm.at[idx_v.at[j]],
                    rows_v.at[pl.ds(j * VEC, VEC)],
                    sem,
                )
                for j in range(K)
            ]
            for c in copies:
                c.wait()
            pltpu.sync_copy(
                rows_v, out_hbm.at[pl.ds(row0 + i * slab_rows, slab_rows)]
            )
            return _

        lax.fori_loop(0, iters, body, None)

    out = gather_kernel(table, idx2d)
    return out.reshape(B, T, D)


# speedup vs baseline: 4.2555x; 1.0291x over previous
import functools

import jax
import jax.numpy as jnp
from jax import lax
from jax.experimental import pallas as pl
from jax.experimental.pallas import tpu as pltpu
from jax.experimental.pallas import tpu_sc as plsc

VEC = 128
K = 2
NB = 4


def kernel(lookup, table):
    B, T = lookup.shape
    V, D = table.shape
    N = B * T

    info = plsc.get_sparse_core_info()
    NW = info.num_cores * info.num_subcores
    assert N % (NW * VEC * K) == 0

    vecs_per_w = N // (NW * VEC)
    slabs = vecs_per_w // K
    slab_rows = K * VEC
    assert (slabs - NB) % NB == 0 and slabs > 2 * NB

    idx2d = lookup.reshape(N // VEC, VEC).astype(jnp.int32)

    mesh = plsc.VectorSubcoreMesh(core_axis_name="c", subcore_axis_name="s")

    @functools.partial(
        pl.kernel,
        mesh=mesh,
        out_type=jax.ShapeDtypeStruct((N, D), jnp.float32),
        scratch_types=[
            pltpu.VMEM((vecs_per_w, VEC), jnp.int32),
            *[pltpu.VMEM((slab_rows, D), jnp.float32) for _ in range(NB)],
            *[pltpu.SemaphoreType.DMA for _ in range(2 * NB)],
        ],
        compiler_params=pltpu.CompilerParams(use_tc_tiling_on_sc=False),
    )
    def gather_kernel(table_hbm, idx_hbm, out_hbm, idx_all, *bufs):
        rows = bufs[:NB]
        gsem = bufs[NB:2 * NB]
        osem = bufs[2 * NB:]

        wid = lax.axis_index("s") * info.num_cores + lax.axis_index("c")
        vrow0 = wid * vecs_per_w
        row0 = wid * (vecs_per_w * VEC)

        pltpu.sync_copy(idx_hbm.at[pl.ds(vrow0, vecs_per_w)], idx_all)

        def fire(i, b):
            for j in range(K):
                pltpu.async_copy(
                    table_hbm.at[idx_all.at[i * K + j]],
                    rows[b].at[pl.ds(j * VEC, VEC)],
                    gsem[b],
                )

        def drain_gather(b):
            pltpu.make_async_copy(
                out_hbm.at[pl.ds(0, slab_rows)], rows[b], gsem[b]
            ).wait()

        def start_wb(i, b):
            pltpu.async_copy(
                rows[b], out_hbm.at[pl.ds(row0 + i * slab_rows, slab_rows)],
                osem[b],
            )

        def wait_wb(b):
            pltpu.make_async_copy(
                out_hbm.at[pl.ds(0, slab_rows)], rows[b], osem[b]
            ).wait()

        fire(0, 0)
        for i in range(1, NB):
            fire(i, i)
            drain_gather(i - 1)
            start_wb(i - 1, i - 1)

        def body(t, _):
            base = NB + t * NB
            for k in range(NB):
                i = base + k
                b = k
                wait_wb(b)
                fire(i, b)
                pb = (k - 1) % NB
                drain_gather(pb)
                start_wb(i - 1, pb)
            return _

        lax.fori_loop(0, (slabs - NB) // NB, body, None)

        last_b = (slabs - 1) % NB
        drain_gather(last_b)
        start_wb(slabs - 1, last_b)
        for b in range(NB):
            wait_wb(b)

    out = gather_kernel(table, idx2d)
    return out.reshape(B, T, D)


# speedup vs baseline: 4.2619x; 1.0015x over previous
import functools

import jax
import jax.numpy as jnp
from jax import lax
from jax.experimental import pallas as pl
from jax.experimental.pallas import tpu as pltpu
from jax.experimental.pallas import tpu_sc as plsc

BPS = 2
NB = 4
SPLITS = (0, 104, 200)


def kernel(lookup, table):
    B, T = lookup.shape
    V, D = table.shape

    info = plsc.get_sparse_core_info()
    NW = info.num_cores * info.num_subcores
    assert B % NW == 0
    b_per_w = B // NW
    slabs = b_per_w // BPS
    assert (slabs - NB) % NB == 0

    idx = lookup.astype(jnp.int32)

    mesh = plsc.VectorSubcoreMesh(core_axis_name="c", subcore_axis_name="s")

    @functools.partial(
        pl.kernel,
        mesh=mesh,
        out_type=jax.ShapeDtypeStruct((B, T, D), jnp.float32),
        scratch_types=[
            pltpu.VMEM((b_per_w, T), jnp.int32),
            *[pltpu.VMEM((BPS, T, D), jnp.float32) for _ in range(NB)],
            *[pltpu.SemaphoreType.DMA for _ in range(2 * NB)],
        ],
        compiler_params=pltpu.CompilerParams(use_tc_tiling_on_sc=False),
    )
    def gather_kernel(table_hbm, idx_hbm, out_hbm, idx_all, *bufs):
        rows = bufs[:NB]
        gsem = bufs[NB:2 * NB]
        osem = bufs[2 * NB:]

        wid = lax.axis_index("s") * info.num_cores + lax.axis_index("c")
        b0 = wid * b_per_w

        pltpu.sync_copy(idx_hbm.at[pl.ds(b0, b_per_w)], idx_all)

        def fire(i, b):
            for kb in range(BPS):
                for lo, hi in zip(SPLITS[:-1], SPLITS[1:]):
                    pltpu.async_copy(
                        table_hbm.at[idx_all.at[i * BPS + kb,
                                                pl.ds(lo, hi - lo)]],
                        rows[b].at[kb, pl.ds(lo, hi - lo)],
                        gsem[b],
                    )

        def drain_gather(b):
            for kb in range(BPS):
                for lo, hi in zip(SPLITS[:-1], SPLITS[1:]):
                    pltpu.make_async_copy(
                        table_hbm.at[pl.ds(0, hi - lo)],
                        rows[b].at[kb, pl.ds(lo, hi - lo)],
                        gsem[b],
                    ).wait()

        def start_wb(i, b):
            pltpu.async_copy(
                rows[b], out_hbm.at[pl.ds(b0 + i * BPS, BPS)], osem[b]
            )

        def wait_wb(b):
            pltpu.make_async_copy(
                out_hbm.at[pl.ds(0, BPS)], rows[b], osem[b]
            ).wait()

        fire(0, 0)
        for i in range(1, NB):
            fire(i, i)
            drain_gather(i - 1)
            start_wb(i - 1, i - 1)

        def body(t, _):
            base = NB + t * NB
            for k in range(NB):
                i = base + k
                wait_wb(k)
                fire(i, k)
                pb = (k - 1) % NB
                drain_gather(pb)
                start_wb(i - 1, pb)
            return _

        lax.fori_loop(0, (slabs - NB) // NB, body, None)

        last_b = (slabs - 1) % NB
        drain_gather(last_b)
        start_wb(slabs - 1, last_b)
        for b in range(NB):
            wait_wb(b)

    return gather_kernel(table, idx)
